# baseline (speedup 1.0000x reference)
import functools

import jax
import jax.numpy as jnp
from jax import lax
from jax.experimental import pallas as pl
from jax.experimental.pallas import tpu as pltpu
from jax.experimental.pallas import tpu_sc as plsc

_N_TOKENS = 8192
_N_CLUSTER = 1024
_EMB = 64

_BT = 1024
_NB = _N_TOKENS // _BT

_NC = 2
_NS = 16
_NW = _NC * _NS
_B1 = _N_TOKENS // _NW
_B2 = _N_CLUSTER // _NW
_CHUNK = 128


def _dist_argmin_kernel(e_ref, r_ref, idxe_ref, idxr_ref, bestv_ref):
    i = pl.program_id(0)

    e = e_ref[...]
    r = r_ref[...]

    m = lax.dot_general(e, r, (((1,), (1,)), ((), ())),
                        preferred_element_type=jnp.float32)
    e2 = jnp.sum(e * e, axis=1)
    r2 = jnp.sum(r * r, axis=1)
    d2 = e2[:, None] + r2[None, :] - 2.0 * m
    d2 = jnp.maximum(d2, 0.0)

    big = jnp.int32(2 ** 30)

    rmin = jnp.min(d2, axis=1, keepdims=True)
    iota_c = lax.broadcasted_iota(jnp.int32, d2.shape, 1)
    ridx = jnp.min(jnp.where(d2 == rmin, iota_c, big), axis=1)
    idxe_ref[0, 0, :] = ridx

    @pl.when(i == 0)
    def _():
        bestv_ref[...] = jnp.full((8, _N_CLUSTER), jnp.inf, jnp.float32)
        idxr_ref[...] = jnp.zeros((8, _N_CLUSTER), jnp.int32)

    cmin = jnp.min(d2, axis=0, keepdims=True)
    iota_r = lax.broadcasted_iota(jnp.int32, d2.shape, 0)
    cidx = jnp.min(jnp.where(d2 == cmin, iota_r, big), axis=0)

    cmin_b = jnp.broadcast_to(cmin, (8, _N_CLUSTER))
    cidx_b = jnp.broadcast_to(cidx[None, :], (8, _N_CLUSTER)) + i * _BT

    cur_v = bestv_ref[...]
    cur_i = idxr_ref[...]
    take = cmin_b < cur_v
    bestv_ref[...] = jnp.where(take, cmin_b, cur_v)
    idxr_ref[...] = jnp.where(take, cidx_b, cur_i)


def _nearest_indices(emb_tensor, repr_tensor):
    idxe, idxr = pl.pallas_call(
        _dist_argmin_kernel,
        grid=(_NB,),
        in_specs=[
            pl.BlockSpec((_BT, _EMB), lambda i: (i, 0)),
            pl.BlockSpec((_N_CLUSTER, _EMB), lambda i: (0, 0)),
        ],
        out_specs=[
            pl.BlockSpec((1, 1, _BT), lambda i: (i, 0, 0)),
            pl.BlockSpec((8, _N_CLUSTER), lambda i: (0, 0)),
        ],
        out_shape=[
            jax.ShapeDtypeStruct((_NB, 1, _BT), jnp.int32),
            jax.ShapeDtypeStruct((8, _N_CLUSTER), jnp.int32),
        ],
        scratch_shapes=[pltpu.VMEM((8, _N_CLUSTER), jnp.float32)],
    )(emb_tensor, repr_tensor)
    return idxe.reshape(_N_TOKENS), idxr[0]


def _sc_gather_body(r_hbm, e_hbm, idxe_hbm, idxr_hbm, xrec_hbm, reprx_hbm,
                    idx1_v, rows1_v, idx2_v, rows2_v, sem):
    wid = lax.axis_index("s") * _NC + lax.axis_index("c")
    base1 = wid * _B1
    base2 = wid * _B2

    pltpu.sync_copy(idxe_hbm.at[wid], idx1_v)
    pltpu.sync_copy(idxr_hbm.at[wid], idx2_v)

    copies = []
    for c in range(_B1 // _CHUNK):
        copies.append(pltpu.async_copy(
            r_hbm.at[idx1_v.at[c]], rows1_v.at[pl.ds(c * _CHUNK, _CHUNK)], sem))
    copies.append(pltpu.async_copy(e_hbm.at[idx2_v], rows2_v, sem))
    for cp in copies:
        cp.wait()

    pltpu.sync_copy(rows1_v, xrec_hbm.at[pl.ds(base1, _B1)])
    pltpu.sync_copy(rows2_v, reprx_hbm.at[pl.ds(base2, _B2)])


@functools.cache
def _make_sc_gather():
    return pl.kernel(
        _sc_gather_body,
        out_type=[
            jax.ShapeDtypeStruct((_N_TOKENS, _EMB), jnp.float32),
            jax.ShapeDtypeStruct((_N_CLUSTER, _EMB), jnp.float32),
        ],
        mesh=plsc.VectorSubcoreMesh(core_axis_name="c", subcore_axis_name="s",
                                    num_cores=_NC, num_subcores=_NS),
        scratch_types=[
            pltpu.VMEM((_B1 // _CHUNK, _CHUNK), jnp.int32),
            pltpu.VMEM((_B1, _EMB), jnp.float32),
            pltpu.VMEM((_B2,), jnp.int32),
            pltpu.VMEM((_B2, _EMB), jnp.float32),
            pltpu.SemaphoreType.DMA,
        ],
        compiler_params=pltpu.CompilerParams(use_tc_tiling_on_sc=False),
    )


def kernel(emb_tensor, repr_tensor):
    idx_e, idx_r = _nearest_indices(emb_tensor, repr_tensor)
    idxe_3d = idx_e.reshape(_NW, _B1 // _CHUNK, _CHUNK)
    idxr_2d = idx_r.reshape(_NW, _B2)
    x_recon, repr_x = _make_sc_gather()(repr_tensor, emb_tensor, idxe_3d, idxr_2d)
    return (x_recon, emb_tensor, repr_tensor, repr_x)


# speedup vs baseline: 1.0806x; 1.0806x over previous
import functools

import jax
import jax.numpy as jnp
from jax import lax
from jax.experimental import pallas as pl
from jax.experimental.pallas import tpu as pltpu
from jax.experimental.pallas import tpu_sc as plsc

_N_TOKENS = 8192
_N_CLUSTER = 1024
_EMB = 64

_BT = 2048
_NB = _N_TOKENS // _BT
_NSTRIP = _N_CLUSTER // 128
_NCH = 8
_C = _BT // _NCH

_NC = 2
_NS = 16
_NW = _NC * _NS
_B1 = _N_TOKENS // _NW
_B2 = _N_CLUSTER // _NW
_CHUNK = 128

def _dist_argmin_kernel(e_ref, r_ref, idxe_ref, idxr_ref, bestv_ref):
    i = pl.program_id(0)
    _BIG = jnp.float32(1e9)

    e = e_ref[...]
    r = r_ref[...]
    e2 = jnp.sum(e * e, axis=1)
    r2 = jnp.sum(r * r, axis=1)

    rv = None
    rsk = None
    col_vals = []
    col_idx = []
    for k in range(_NSTRIP):
        rk = r[k * 128:(k + 1) * 128, :]
        m = lax.dot_general(e, rk, (((1,), (1,)), ((), ())),
                            preferred_element_type=jnp.float32)
        d2 = (e2[:, None] + r2[k * 128:(k + 1) * 128][None, :]) - 2.0 * m

        if rv is None:
            rv = d2
            rsk = jnp.zeros((_BT, 128), jnp.float32)
        else:
            t = d2 < rv
            rsk = jnp.where(t, jnp.float32(k), rsk)
            rv = jnp.where(t, d2, rv)

        cv = d2[0:_C, :]
        cs = jnp.zeros((_C, 128), jnp.float32)
        for s in range(1, _NCH):
            ch = d2[s * _C:(s + 1) * _C, :]
            ts = ch < cv
            cs = jnp.where(ts, jnp.float32(s), cs)
            cv = jnp.where(ts, ch, cv)
        cmin = jnp.min(cv, axis=0, keepdims=True)
        rowfull = cs * jnp.float32(_C) + lax.broadcasted_iota(
            jnp.int32, (_C, 128), 0).astype(jnp.float32)
        cidx = jnp.min(jnp.where(cv == cmin, rowfull, _BIG),
                       axis=0, keepdims=True)
        col_vals.append(cmin)
        col_idx.append(cidx)

    rmin = jnp.min(rv, axis=1, keepdims=True)
    jfull = rsk * jnp.float32(128.0) + lax.broadcasted_iota(
        jnp.int32, (_BT, 128), 1).astype(jnp.float32)
    ridx = jnp.min(jnp.where(rv == rmin, jfull, _BIG), axis=1)
    idxe_ref[0, 0, :] = ridx

    cmin_all = jnp.concatenate(col_vals, axis=1)
    cidx_all = jnp.concatenate(col_idx, axis=1) + jnp.float32(i) * _BT

    @pl.when(i == 0)
    def _():
        bestv_ref[...] = jnp.full((1, _N_CLUSTER), _BIG, jnp.float32)
        idxr_ref[...] = jnp.zeros((1, _N_CLUSTER), jnp.float32)

    cur_v = bestv_ref[...]
    cur_i = idxr_ref[...]
    take = cmin_all < cur_v
    bestv_ref[...] = jnp.where(take, cmin_all, cur_v)
    idxr_ref[...] = jnp.where(take, cidx_all, cur_i)


def _nearest_indices(emb_tensor, repr_tensor):
    idxe, idxr = pl.pallas_call(
        _dist_argmin_kernel,
        grid=(_NB,),
        in_specs=[
            pl.BlockSpec((_BT, _EMB), lambda i: (i, 0)),
            pl.BlockSpec((_N_CLUSTER, _EMB), lambda i: (0, 0)),
        ],
        out_specs=[
            pl.BlockSpec((1, 1, _BT), lambda i: (i, 0, 0)),
            pl.BlockSpec((1, _N_CLUSTER), lambda i: (0, 0)),
        ],
        out_shape=[
            jax.ShapeDtypeStruct((_NB, 1, _BT), jnp.float32),
            jax.ShapeDtypeStruct((1, _N_CLUSTER), jnp.float32),
        ],
        scratch_shapes=[pltpu.VMEM((1, _N_CLUSTER), jnp.float32)],
    )(emb_tensor, repr_tensor)
    return idxe.reshape(_N_TOKENS).astype(jnp.int32), idxr[0].astype(jnp.int32)


def _sc_gather_body(r_hbm, e_hbm, idxe_hbm, idxr_hbm, xrec_hbm, reprx_hbm,
                    idx1_v, rows1_v, idx2_v, rows2_v, semi, sem, semo):
    wid = lax.axis_index("s") * _NC + lax.axis_index("c")
    base1 = wid * _B1
    base2 = wid * _B2

    ld1 = pltpu.async_copy(idxe_hbm.at[wid], idx1_v, semi)
    ld2 = pltpu.async_copy(idxr_hbm.at[wid], idx2_v, semi)
    ld1.wait()
    ld2.wait()

    copies = []
    for c in range(_B1 // _CHUNK):
        copies.append(pltpu.async_copy(
            r_hbm.at[idx1_v.at[c]], rows1_v.at[pl.ds(c * _CHUNK, _CHUNK)], sem))
    copies.append(pltpu.async_copy(e_hbm.at[idx2_v], rows2_v, sem))
    for cp in copies:
        cp.wait()

    st1 = pltpu.async_copy(rows1_v, xrec_hbm.at[pl.ds(base1, _B1)], semo)
    st2 = pltpu.async_copy(rows2_v, reprx_hbm.at[pl.ds(base2, _B2)], semo)
    st1.wait()
    st2.wait()


@functools.cache
def _make_sc_gather():
    return pl.kernel(
        _sc_gather_body,
        out_type=[
            jax.ShapeDtypeStruct((_N_TOKENS, _EMB), jnp.float32),
            jax.ShapeDtypeStruct((_N_CLUSTER, _EMB), jnp.float32),
        ],
        mesh=plsc.VectorSubcoreMesh(core_axis_name="c", subcore_axis_name="s",
                                    num_cores=_NC, num_subcores=_NS),
        scratch_types=[
            pltpu.VMEM((_B1 // _CHUNK, _CHUNK), jnp.int32),
            pltpu.VMEM((_B1, _EMB), jnp.float32),
            pltpu.VMEM((_B2,), jnp.int32),
            pltpu.VMEM((_B2, _EMB), jnp.float32),
            pltpu.SemaphoreType.DMA,
            pltpu.SemaphoreType.DMA,
            pltpu.SemaphoreType.DMA,
        ],
        compiler_params=pltpu.CompilerParams(use_tc_tiling_on_sc=False),
    )


def kernel(emb_tensor, repr_tensor):
    idx_e, idx_r = _nearest_indices(emb_tensor, repr_tensor)
    idxe_3d = idx_e.reshape(_NW, _B1 // _CHUNK, _CHUNK)
    idxr_2d = idx_r.reshape(_NW, _B2)
    x_recon, repr_x = _make_sc_gather()(repr_tensor, emb_tensor, idxe_3d, idxr_2d)
    return (x_recon, emb_tensor, repr_tensor, repr_x)


# speedup vs baseline: 1.1133x; 1.0302x over previous
import functools

import jax
import jax.numpy as jnp
from jax import lax
from jax.experimental import pallas as pl
from jax.experimental.pallas import tpu as pltpu
from jax.experimental.pallas import tpu_sc as plsc

_N_TOKENS = 8192
_N_CLUSTER = 1024
_EMB = 64

_BT = 2048
_NB = _N_TOKENS // _BT
_NSTRIP = _N_CLUSTER // 128
_NCH = 8
_C = _BT // _NCH

_NC = 2
_NS = 16
_NW = _NC * _NS
_B1 = _N_TOKENS // _NW
_B2 = _N_CLUSTER // _NW
_CHUNK = 32

def _dist_argmin_kernel(e_ref, r_ref, idxe_ref, idxr_ref, bestv_ref, besti_ref):
    i = pl.program_id(0)
    _BIG = jnp.float32(1e9)

    e = e_ref[...]
    r = r_ref[...]
    e2 = jnp.sum(e * e, axis=1)
    r2 = jnp.sum(r * r, axis=1)

    rv = None
    rsk = None
    col_vals = []
    col_idx = []
    for k in range(_NSTRIP):
        rk = r[k * 128:(k + 1) * 128, :]
        m = lax.dot_general(e, rk, (((1,), (1,)), ((), ())),
                            preferred_element_type=jnp.float32)
        d2 = (e2[:, None] + r2[k * 128:(k + 1) * 128][None, :]) - 2.0 * m

        if rv is None:
            rv = d2
            rsk = jnp.zeros((_BT, 128), jnp.float32)
        else:
            t = d2 < rv
            rsk = jnp.where(t, jnp.float32(k), rsk)
            rv = jnp.where(t, d2, rv)

        cv = d2[0:_C, :]
        cs = jnp.zeros((_C, 128), jnp.float32)
        for s in range(1, _NCH):
            ch = d2[s * _C:(s + 1) * _C, :]
            ts = ch < cv
            cs = jnp.where(ts, jnp.float32(s), cs)
            cv = jnp.where(ts, ch, cv)
        cmin = jnp.min(cv, axis=0, keepdims=True)
        rowfull = cs * jnp.float32(_C) + lax.broadcasted_iota(
            jnp.int32, (_C, 128), 0).astype(jnp.float32)
        cidx = jnp.min(jnp.where(cv == cmin, rowfull, _BIG),
                       axis=0, keepdims=True)
        col_vals.append(cmin)
        col_idx.append(cidx)

    rmin = jnp.min(rv, axis=1, keepdims=True)
    jfull = rsk * jnp.float32(128.0) + lax.broadcasted_iota(
        jnp.int32, (_BT, 128), 1).astype(jnp.float32)
    ridx = jnp.min(jnp.where(rv == rmin, jfull, _BIG), axis=1)
    idxe_ref[0, 0, :] = ridx.astype(jnp.int32)

    cmin_all = jnp.concatenate(col_vals, axis=1)
    cidx_all = jnp.concatenate(col_idx, axis=1) + jnp.float32(i) * _BT

    @pl.when(i == 0)
    def _():
        bestv_ref[...] = jnp.full((1, _N_CLUSTER), _BIG, jnp.float32)
        besti_ref[...] = jnp.zeros((1, _N_CLUSTER), jnp.float32)

    cur_v = bestv_ref[...]
    cur_i = besti_ref[...]
    take = cmin_all < cur_v
    bestv_ref[...] = jnp.where(take, cmin_all, cur_v)
    best_i = jnp.where(take, cidx_all, cur_i)
    besti_ref[...] = best_i
    idxr_ref[...] = best_i.astype(jnp.int32)


def _nearest_indices(emb_tensor, repr_tensor):
    idxe, idxr = pl.pallas_call(
        _dist_argmin_kernel,
        grid=(_NB,),
        in_specs=[
            pl.BlockSpec((_BT, _EMB), lambda i: (i, 0)),
            pl.BlockSpec((_N_CLUSTER, _EMB), lambda i: (0, 0)),
        ],
        out_specs=[
            pl.BlockSpec((1, 1, _BT), lambda i: (i, 0, 0)),
            pl.BlockSpec((1, _N_CLUSTER), lambda i: (0, 0)),
        ],
        out_shape=[
            jax.ShapeDtypeStruct((_NB, 1, _BT), jnp.int32),
            jax.ShapeDtypeStruct((1, _N_CLUSTER), jnp.int32),
        ],
        scratch_shapes=[pltpu.VMEM((1, _N_CLUSTER), jnp.float32),
                        pltpu.VMEM((1, _N_CLUSTER), jnp.float32)],
    )(emb_tensor, repr_tensor)
    return idxe.reshape(_N_TOKENS), idxr[0]


def _sc_gather_body(r_hbm, e_hbm, idxe_hbm, idxr_hbm, xrec_hbm, reprx_hbm,
                    idx1_v, rows1_v, idx2_v, rows2_v, semi, sem, semo):
    wid = lax.axis_index("s") * _NC + lax.axis_index("c")
    base1 = wid * _B1
    base2 = wid * _B2

    ld1 = pltpu.async_copy(idxe_hbm.at[wid], idx1_v, semi)
    ld2 = pltpu.async_copy(idxr_hbm.at[wid], idx2_v, semi)
    ld1.wait()
    ld2.wait()

    copies = []
    for c in range(_B1 // _CHUNK):
        copies.append(pltpu.async_copy(
            r_hbm.at[idx1_v.at[c]], rows1_v.at[pl.ds(c * _CHUNK, _CHUNK)], sem))
    copies.append(pltpu.async_copy(e_hbm.at[idx2_v], rows2_v, sem))
    for cp in copies:
        cp.wait()

    st1 = pltpu.async_copy(rows1_v, xrec_hbm.at[pl.ds(base1, _B1)], semo)
    st2 = pltpu.async_copy(rows2_v, reprx_hbm.at[pl.ds(base2, _B2)], semo)
    st1.wait()
    st2.wait()


@functools.cache
def _make_sc_gather():
    return pl.kernel(
        _sc_gather_body,
        out_type=[
            jax.ShapeDtypeStruct((_N_TOKENS, _EMB), jnp.float32),
            jax.ShapeDtypeStruct((_N_CLUSTER, _EMB), jnp.float32),
        ],
        mesh=plsc.VectorSubcoreMesh(core_axis_name="c", subcore_axis_name="s",
                                    num_cores=_NC, num_subcores=_NS),
        scratch_types=[
            pltpu.VMEM((_B1 // _CHUNK, _CHUNK), jnp.int32),
            pltpu.VMEM((_B1, _EMB), jnp.float32),
            pltpu.VMEM((_B2,), jnp.int32),
            pltpu.VMEM((_B2, _EMB), jnp.float32),
            pltpu.SemaphoreType.DMA,
            pltpu.SemaphoreType.DMA,
            pltpu.SemaphoreType.DMA,
        ],
        compiler_params=pltpu.CompilerParams(use_tc_tiling_on_sc=False),
    )


def kernel(emb_tensor, repr_tensor):
    idx_e, idx_r = _nearest_indices(emb_tensor, repr_tensor)
    idxe_3d = idx_e.reshape(_NW, _B1 // _CHUNK, _CHUNK)
    idxr_2d = idx_r.reshape(_NW, _B2)
    x_recon, repr_x = _make_sc_gather()(repr_tensor, emb_tensor, idxe_3d, idxr_2d)
    return (x_recon, emb_tensor, repr_tensor, repr_x)


# speedup vs baseline: 1.2176x; 1.0937x over previous
import functools

import jax
import jax.numpy as jnp
from jax import lax
from jax.experimental import pallas as pl
from jax.experimental.pallas import tpu as pltpu
from jax.experimental.pallas import tpu_sc as plsc

_N_TOKENS = 8192
_N_CLUSTER = 1024
_EMB = 64

_BT = 2048
_NB = _N_TOKENS // _BT
_NSTRIP = _N_CLUSTER // 128
_NCH = 8
_C = _BT // _NCH

_NC = 2
_NS = 16
_NW = _NC * _NS
_B1 = _N_TOKENS // _NW
_B2 = _N_CLUSTER // _NW
_CHUNK = 32

def _dist_argmin_kernel(e_ref, r_ref, idxe_ref, idxr_ref, cv_ref, ci_ref):
    i = pl.program_id(0)
    _BIG = jnp.float32(1e9)
    f32 = jnp.float32

    e = e_ref[...]
    r = r_ref[...]
    e2 = jnp.sum(e * e, axis=1, keepdims=True)
    r2 = jnp.sum(r * r, axis=1, keepdims=True)
    e2row = e2.reshape(1, _BT)

    m2 = lax.dot_general(r * f32(-2.0), e, (((1,), (1,)), ((), ())),
                         preferred_element_type=f32)
    d2 = (r2 + e2row) + m2

    rv = d2[0:128, :]
    cs = jnp.zeros((128, _BT), f32)
    for s in range(1, _N_CLUSTER // 128):
        ch = d2[s * 128:(s + 1) * 128, :]
        t = ch < rv
        cs = jnp.where(t, f32(s), cs)
        rv = jnp.where(t, ch, rv)
    rmin = jnp.min(rv, axis=0, keepdims=True)
    jfull = cs * f32(128.0) + lax.broadcasted_iota(
        jnp.int32, (128, _BT), 0).astype(f32)
    ridx = jnp.min(jnp.where(rv == rmin, jfull, _BIG), axis=0)
    idxe_ref[0, 0, :] = ridx.astype(jnp.int32)

    @pl.when(i == 0)
    def _():
        cv_ref[...] = jnp.full((_N_CLUSTER, 128), _BIG, f32)
        ci_ref[...] = jnp.zeros((_N_CLUSTER, 128), f32)

    cv = cv_ref[...]
    ci = ci_ref[...]
    for g in range(_BT // 128):
        blk = d2[:, g * 128:(g + 1) * 128]
        t = blk < cv
        ci = jnp.where(t, f32(i * (_BT // 128) + g), ci)
        cv = jnp.where(t, blk, cv)
    cv_ref[...] = cv
    ci_ref[...] = ci

    @pl.when(i == _NB - 1)
    def _():
        cmin = jnp.min(cv, axis=1, keepdims=True)
        tfull = ci * f32(128.0) + lax.broadcasted_iota(
            jnp.int32, (_N_CLUSTER, 128), 1).astype(f32)
        cidx = jnp.min(jnp.where(cv == cmin, tfull, _BIG), axis=1)
        idxr_ref[...] = cidx.astype(jnp.int32).reshape(1, _N_CLUSTER)


def _nearest_indices(emb_tensor, repr_tensor):
    idxe, idxr = pl.pallas_call(
        _dist_argmin_kernel,
        grid=(_NB,),
        in_specs=[
            pl.BlockSpec((_BT, _EMB), lambda i: (i, 0)),
            pl.BlockSpec((_N_CLUSTER, _EMB), lambda i: (0, 0)),
        ],
        out_specs=[
            pl.BlockSpec((1, 1, _BT), lambda i: (i, 0, 0)),
            pl.BlockSpec((1, _N_CLUSTER), lambda i: (0, 0)),
        ],
        out_shape=[
            jax.ShapeDtypeStruct((_NB, 1, _BT), jnp.int32),
            jax.ShapeDtypeStruct((1, _N_CLUSTER), jnp.int32),
        ],
        scratch_shapes=[pltpu.VMEM((_N_CLUSTER, 128), jnp.float32),
                        pltpu.VMEM((_N_CLUSTER, 128), jnp.float32)],
    )(emb_tensor, repr_tensor)
    return idxe.reshape(_N_TOKENS), idxr[0]


def _sc_gather_body(r_hbm, e_hbm, idxe_hbm, idxr_hbm, xrec_hbm, reprx_hbm,
                    idx1_v, rows1_v, idx2_v, rows2_v, semi, sem, semo):
    wid = lax.axis_index("s") * _NC + lax.axis_index("c")
    base1 = wid * _B1
    base2 = wid * _B2

    ld1 = pltpu.async_copy(idxe_hbm.at[wid], idx1_v, semi)
    ld2 = pltpu.async_copy(idxr_hbm.at[wid], idx2_v, semi)
    ld1.wait()
    ld2.wait()

    copies = []
    for c in range(_B1 // _CHUNK):
        copies.append(pltpu.async_copy(
            r_hbm.at[idx1_v.at[c]], rows1_v.at[pl.ds(c * _CHUNK, _CHUNK)], sem))
    copies.append(pltpu.async_copy(e_hbm.at[idx2_v], rows2_v, sem))
    for cp in copies:
        cp.wait()

    st1 = pltpu.async_copy(rows1_v, xrec_hbm.at[pl.ds(base1, _B1)], semo)
    st2 = pltpu.async_copy(rows2_v, reprx_hbm.at[pl.ds(base2, _B2)], semo)
    st1.wait()
    st2.wait()


@functools.cache
def _make_sc_gather():
    return pl.kernel(
        _sc_gather_body,
        out_type=[
            jax.ShapeDtypeStruct((_N_TOKENS, _EMB), jnp.float32),
            jax.ShapeDtypeStruct((_N_CLUSTER, _EMB), jnp.float32),
        ],
        mesh=plsc.VectorSubcoreMesh(core_axis_name="c", subcore_axis_name="s",
                                    num_cores=_NC, num_subcores=_NS),
        scratch_types=[
            pltpu.VMEM((_B1 // _CHUNK, _CHUNK), jnp.int32),
            pltpu.VMEM((_B1, _EMB), jnp.float32),
            pltpu.VMEM((_B2,), jnp.int32),
            pltpu.VMEM((_B2, _EMB), jnp.float32),
            pltpu.SemaphoreType.DMA,
            pltpu.SemaphoreType.DMA,
            pltpu.SemaphoreType.DMA,
        ],
        compiler_params=pltpu.CompilerParams(use_tc_tiling_on_sc=False),
    )


def kernel(emb_tensor, repr_tensor):
    idx_e, idx_r = _nearest_indices(emb_tensor, repr_tensor)
    idxe_3d = idx_e.reshape(_NW, _B1 // _CHUNK, _CHUNK)
    idxr_2d = idx_r.reshape(_NW, _B2)
    x_recon, repr_x = _make_sc_gather()(repr_tensor, emb_tensor, idxe_3d, idxr_2d)
    return (x_recon, emb_tensor, repr_tensor, repr_x)


# speedup vs baseline: 1.4696x; 1.2070x over previous
import functools

import jax
import jax.numpy as jnp
from jax import lax
from jax.experimental import pallas as pl
from jax.experimental.pallas import tpu as pltpu
from jax.experimental.pallas import tpu_sc as plsc

_N_TOKENS = 8192
_N_CLUSTER = 1024
_EMB = 64

_BT = 2048
_NB = _N_TOKENS // _BT
_NSTRIP = _N_CLUSTER // 128
_NCH = 8
_C = _BT // _NCH

_NC = 2
_NS = 16
_NW = _NC * _NS
_B1 = _N_TOKENS // _NW
_B2 = _N_CLUSTER // _NW
_CHUNK = 32

def _dist_argmin_kernel(e_ref, r_ref, idxe_ref, idxr_ref, cv_ref, ci_ref):
    i = pl.program_id(0)
    _BIG = jnp.float32(1e9)
    f32 = jnp.float32

    e = e_ref[...]
    r = r_ref[...]
    e2 = jnp.sum(e * e, axis=1, keepdims=True)
    r2 = jnp.sum(r * r, axis=1, keepdims=True)
    e2row = e2.reshape(1, _BT)

    m2 = lax.dot_general(r * f32(-2.0), e, (((1,), (1,)), ((), ())),
                         preferred_element_type=f32)
    d2 = (r2 + e2row) + m2

    rv = d2[0:128, :]
    cs = jnp.zeros((128, _BT), f32)
    for s in range(1, _N_CLUSTER // 128):
        ch = d2[s * 128:(s + 1) * 128, :]
        t = ch < rv
        cs = jnp.where(t, f32(s), cs)
        rv = jnp.where(t, ch, rv)
    rmin = jnp.min(rv, axis=0, keepdims=True)
    jfull = cs * f32(128.0) + lax.broadcasted_iota(
        jnp.int32, (128, _BT), 0).astype(f32)
    ridx = jnp.min(jnp.where(rv == rmin, jfull, _BIG), axis=0)
    idxe_ref[0, 0, :] = ridx.astype(jnp.int32)

    @pl.when(i == 0)
    def _():
        cv_ref[...] = jnp.full((_N_CLUSTER, 128), _BIG, f32)
        ci_ref[...] = jnp.zeros((_N_CLUSTER, 128), f32)

    cv = cv_ref[...]
    ci = ci_ref[...]
    for g in range(_BT // 128):
        blk = d2[:, g * 128:(g + 1) * 128]
        t = blk < cv
        ci = jnp.where(t, f32(i * (_BT // 128) + g), ci)
        cv = jnp.where(t, blk, cv)
    cv_ref[...] = cv
    ci_ref[...] = ci

    @pl.when(i == _NB - 1)
    def _():
        cmin = jnp.min(cv, axis=1, keepdims=True)
        tfull = ci * f32(128.0) + lax.broadcasted_iota(
            jnp.int32, (_N_CLUSTER, 128), 1).astype(f32)
        cidx = jnp.min(jnp.where(cv == cmin, tfull, _BIG), axis=1)
        idxr_ref[...] = cidx.astype(jnp.int32).reshape(1, _N_CLUSTER)


def _nearest_indices(emb_tensor, repr_tensor):
    idxe, idxr = pl.pallas_call(
        _dist_argmin_kernel,
        grid=(_NB,),
        in_specs=[
            pl.BlockSpec((_BT, _EMB), lambda i: (i, 0)),
            pl.BlockSpec((_N_CLUSTER, _EMB), lambda i: (0, 0)),
        ],
        out_specs=[
            pl.BlockSpec((1, 1, _BT), lambda i: (i, 0, 0)),
            pl.BlockSpec((1, _N_CLUSTER), lambda i: (0, 0)),
        ],
        out_shape=[
            jax.ShapeDtypeStruct((_NB, 1, _BT), jnp.int32),
            jax.ShapeDtypeStruct((1, _N_CLUSTER), jnp.int32),
        ],
        scratch_shapes=[pltpu.VMEM((_N_CLUSTER, 128), jnp.float32),
                        pltpu.VMEM((_N_CLUSTER, 128), jnp.float32)],
    )(emb_tensor, repr_tensor)
    return idxe.reshape(_N_TOKENS), idxr[0]


def _sc_gather_body(r_hbm, e_hbm, idxe_hbm, idxr_hbm, xrec_hbm, reprx_hbm,
                    r_sh, e_sh, idx1_v, rows1_v, idx2_v, rows2_v,
                    semi, sem, semo):
    sid = lax.axis_index("s")
    wid = sid * _NC + lax.axis_index("c")
    base1 = wid * _B1
    base2 = wid * _B2

    ld1 = pltpu.async_copy(idxe_hbm.at[wid], idx1_v, semi)
    ld2 = pltpu.async_copy(idxr_hbm.at[wid], idx2_v, semi)

    er = _N_TOKENS // _NS
    rr = _N_CLUSTER // _NS
    pltpu.sync_copy(e_hbm.at[pl.ds(sid * er, er)], e_sh.at[pl.ds(sid * er, er)])
    pltpu.sync_copy(r_hbm.at[pl.ds(sid * rr, rr)], r_sh.at[pl.ds(sid * rr, rr)])
    plsc.subcore_barrier()

    ld1.wait()
    ld2.wait()

    copies = []
    for c in range(_B1 // _CHUNK):
        copies.append(pltpu.async_copy(
            r_sh.at[idx1_v.at[c]], rows1_v.at[pl.ds(c * _CHUNK, _CHUNK)], sem))
    copies.append(pltpu.async_copy(e_sh.at[idx2_v], rows2_v, sem))
    for cp in copies:
        cp.wait()

    st1 = pltpu.async_copy(rows1_v, xrec_hbm.at[pl.ds(base1, _B1)], semo)
    st2 = pltpu.async_copy(rows2_v, reprx_hbm.at[pl.ds(base2, _B2)], semo)
    st1.wait()
    st2.wait()


@functools.cache
def _make_sc_gather():
    return pl.kernel(
        _sc_gather_body,
        out_type=[
            jax.ShapeDtypeStruct((_N_TOKENS, _EMB), jnp.float32),
            jax.ShapeDtypeStruct((_N_CLUSTER, _EMB), jnp.float32),
        ],
        mesh=plsc.VectorSubcoreMesh(core_axis_name="c", subcore_axis_name="s",
                                    num_cores=_NC, num_subcores=_NS),
        scratch_types=[
            pltpu.VMEM_SHARED((_N_CLUSTER, _EMB), jnp.float32),
            pltpu.VMEM_SHARED((_N_TOKENS, _EMB), jnp.float32),
            pltpu.VMEM((_B1 // _CHUNK, _CHUNK), jnp.int32),
            pltpu.VMEM((_B1, _EMB), jnp.float32),
            pltpu.VMEM((_B2,), jnp.int32),
            pltpu.VMEM((_B2, _EMB), jnp.float32),
            pltpu.SemaphoreType.DMA,
            pltpu.SemaphoreType.DMA,
            pltpu.SemaphoreType.DMA,
        ],
        compiler_params=pltpu.CompilerParams(use_tc_tiling_on_sc=False),
    )


def kernel(emb_tensor, repr_tensor):
    idx_e, idx_r = _nearest_indices(emb_tensor, repr_tensor)
    idxe_3d = idx_e.reshape(_NW, _B1 // _CHUNK, _CHUNK)
    idxr_2d = idx_r.reshape(_NW, _B2)
    x_recon, repr_x = _make_sc_gather()(repr_tensor, emb_tensor, idxe_3d, idxr_2d)
    return (x_recon, emb_tensor, repr_tensor, repr_x)


# speedup vs baseline: 1.4908x; 1.0145x over previous
import functools

import jax
import jax.numpy as jnp
from jax import lax
from jax.experimental import pallas as pl
from jax.experimental.pallas import tpu as pltpu
from jax.experimental.pallas import tpu_sc as plsc

_N_TOKENS = 8192
_N_CLUSTER = 1024
_EMB = 64

_BT = 2048
_NB = _N_TOKENS // _BT
_NSTRIP = _N_CLUSTER // 128
_NCH = 8
_C = _BT // _NCH

_NC = 2
_NS = 16
_NW = _NC * _NS
_B1 = _N_TOKENS // _NW
_B2 = _N_CLUSTER // _NW
_CHUNK = 32

def _dist_argmin_kernel(e_ref, r_ref, idxe_ref, idxr_ref, cv_ref, ci_ref):
    i = pl.program_id(0)
    _BIG = jnp.float32(1e9)
    f32 = jnp.float32

    e = e_ref[...]
    r = r_ref[...]
    e2 = jnp.sum(e * e, axis=1, keepdims=True)
    r2 = jnp.sum(r * r, axis=1, keepdims=True)
    e2row = e2.reshape(1, _BT)

    m2 = lax.dot_general(r * f32(-2.0), e, (((1,), (1,)), ((), ())),
                         preferred_element_type=f32)
    d2 = (r2 + e2row) + m2

    rv = d2[0:128, :]
    cs = jnp.zeros((128, _BT), f32)
    for s in range(1, _N_CLUSTER // 128):
        ch = d2[s * 128:(s + 1) * 128, :]
        cs = jnp.where(ch < rv, f32(s), cs)
        rv = jnp.minimum(rv, ch)
    rmin = jnp.min(rv, axis=0, keepdims=True)
    jfull = cs * f32(128.0) + lax.broadcasted_iota(
        jnp.int32, (128, _BT), 0).astype(f32)
    ridx = jnp.min(jnp.where(rv == rmin, jfull, _BIG), axis=0)
    idxe_ref[0, 0, :] = ridx.astype(jnp.int32)

    @pl.when(i == 0)
    def _():
        cv_ref[...] = jnp.full((_N_CLUSTER, 128), _BIG, f32)
        ci_ref[...] = jnp.zeros((_N_CLUSTER, 128), f32)

    cv = cv_ref[...]
    ci = ci_ref[...]
    for g in range(_BT // 128):
        blk = d2[:, g * 128:(g + 1) * 128]
        ci = jnp.where(blk < cv, f32(i * (_BT // 128) + g), ci)
        cv = jnp.minimum(cv, blk)
    cv_ref[...] = cv
    ci_ref[...] = ci

    @pl.when(i == _NB - 1)
    def _():
        cmin = jnp.min(cv, axis=1, keepdims=True)
        tfull = ci * f32(128.0) + lax.broadcasted_iota(
            jnp.int32, (_N_CLUSTER, 128), 1).astype(f32)
        cidx = jnp.min(jnp.where(cv == cmin, tfull, _BIG), axis=1)
        idxr_ref[...] = cidx.astype(jnp.int32).reshape(1, _N_CLUSTER)


def _nearest_indices(emb_tensor, repr_tensor):
    idxe, idxr = pl.pallas_call(
        _dist_argmin_kernel,
        grid=(_NB,),
        in_specs=[
            pl.BlockSpec((_BT, _EMB), lambda i: (i, 0)),
            pl.BlockSpec((_N_CLUSTER, _EMB), lambda i: (0, 0)),
        ],
        out_specs=[
            pl.BlockSpec((1, 1, _BT), lambda i: (i, 0, 0)),
            pl.BlockSpec((1, _N_CLUSTER), lambda i: (0, 0)),
        ],
        out_shape=[
            jax.ShapeDtypeStruct((_NB, 1, _BT), jnp.int32),
            jax.ShapeDtypeStruct((1, _N_CLUSTER), jnp.int32),
        ],
        scratch_shapes=[pltpu.VMEM((_N_CLUSTER, 128), jnp.float32),
                        pltpu.VMEM((_N_CLUSTER, 128), jnp.float32)],
    )(emb_tensor, repr_tensor)
    return idxe.reshape(_N_TOKENS), idxr[0]


def _sc_gather_body(r_hbm, e_hbm, idxe_hbm, idxr_hbm, xrec_hbm, reprx_hbm,
                    r_sh, e_sh, idx1_v, rows1_v, idx2_v, rows2_v,
                    semi, sem, semo):
    sid = lax.axis_index("s")
    wid = sid * _NC + lax.axis_index("c")
    base1 = wid * _B1
    base2 = wid * _B2

    ld1 = pltpu.async_copy(idxe_hbm.at[wid], idx1_v, semi)
    ld2 = pltpu.async_copy(idxr_hbm.at[wid], idx2_v, semi)

    er = _N_TOKENS // _NS
    rr = _N_CLUSTER // _NS
    pltpu.sync_copy(e_hbm.at[pl.ds(sid * er, er)], e_sh.at[pl.ds(sid * er, er)])
    pltpu.sync_copy(r_hbm.at[pl.ds(sid * rr, rr)], r_sh.at[pl.ds(sid * rr, rr)])
    plsc.subcore_barrier()

    ld1.wait()
    ld2.wait()

    copies = []
    for c in range(_B1 // _CHUNK):
        copies.append(pltpu.async_copy(
            r_sh.at[idx1_v.at[c]], rows1_v.at[pl.ds(c * _CHUNK, _CHUNK)], sem))
    copies.append(pltpu.async_copy(e_sh.at[idx2_v], rows2_v, sem))
    for cp in copies:
        cp.wait()

    st1 = pltpu.async_copy(rows1_v, xrec_hbm.at[pl.ds(base1, _B1)], semo)
    st2 = pltpu.async_copy(rows2_v, reprx_hbm.at[pl.ds(base2, _B2)], semo)
    st1.wait()
    st2.wait()


@functools.cache
def _make_sc_gather():
    return pl.kernel(
        _sc_gather_body,
        out_type=[
            jax.ShapeDtypeStruct((_N_TOKENS, _EMB), jnp.float32),
            jax.ShapeDtypeStruct((_N_CLUSTER, _EMB), jnp.float32),
        ],
        mesh=plsc.VectorSubcoreMesh(core_axis_name="c", subcore_axis_name="s",
                                    num_cores=_NC, num_subcores=_NS),
        scratch_types=[
            pltpu.VMEM_SHARED((_N_CLUSTER, _EMB), jnp.float32),
            pltpu.VMEM_SHARED((_N_TOKENS, _EMB), jnp.float32),
            pltpu.VMEM((_B1 // _CHUNK, _CHUNK), jnp.int32),
            pltpu.VMEM((_B1, _EMB), jnp.float32),
            pltpu.VMEM((_B2,), jnp.int32),
            pltpu.VMEM((_B2, _EMB), jnp.float32),
            pltpu.SemaphoreType.DMA,
            pltpu.SemaphoreType.DMA,
            pltpu.SemaphoreType.DMA,
        ],
        compiler_params=pltpu.CompilerParams(use_tc_tiling_on_sc=False),
    )


def kernel(emb_tensor, repr_tensor):
    idx_e, idx_r = _nearest_indices(emb_tensor, repr_tensor)
    idxe_3d = idx_e.reshape(_NW, _B1 // _CHUNK, _CHUNK)
    idxr_2d = idx_r.reshape(_NW, _B2)
    x_recon, repr_x = _make_sc_gather()(repr_tensor, emb_tensor, idxe_3d, idxr_2d)
    return (x_recon, emb_tensor, repr_tensor, repr_x)
